# baseline (speedup 1.0000x reference)
import functools

import jax
import jax.numpy as jnp
from jax import lax
from jax.experimental import pallas as pl
from jax.experimental.pallas import tpu as pltpu
from jax.experimental.pallas import tpu_sc as plsc

N = 10000
E = 320000
D = 128
DLLM = 640
DPROJ = 112
NSTRUCT = 5
NC = 2
NS = 16
NW = NC * NS
CK = 96
CH = 106
EPAD = NW * CH * CK
NACC = 10112
SEG = NACC // NS
BLK = 1000
GRID = N // BLK


def _sc_deg_body(dst_hbm, deg_out, dst_v, deg_v):
    c = lax.axis_index("c")
    s = lax.axis_index("s")
    wid = c * NS + s
    pltpu.sync_copy(dst_hbm.at[wid], dst_v)

    zero16 = jnp.zeros((16,), jnp.float32)

    def zbody(i, carry):
        deg_v[pl.ds(i * 16, 16)] = zero16
        return carry

    lax.fori_loop(0, NACC // 16, zbody, 0)

    ones16 = jnp.ones((16,), jnp.float32)

    def ebody(k, carry):
        idx = dst_v[pl.ds(k * 16, 16)]
        plsc.addupdate_scatter(deg_v, [idx], ones16)
        return carry

    lax.fori_loop(0, (CH * CK) // 16, ebody, 0)
    pltpu.sync_copy(deg_v, deg_out.at[wid])


_sc_deg = pl.kernel(
    _sc_deg_body,
    out_type=jax.ShapeDtypeStruct((NW, NACC), jnp.float32),
    mesh=plsc.VectorSubcoreMesh(core_axis_name="c", subcore_axis_name="s"),
    scratch_types=[
        pltpu.VMEM((CH * CK,), jnp.int32),
        pltpu.VMEM((NACC,), jnp.float32),
    ],
    compiler_params=pltpu.CompilerParams(needs_layout_passes=False),
)


def _sc_scatter_body(h_hbm, src_hbm, dst_hbm, z_hbm, out_hbm,
                     sidx_v, didx_v, rows_v, acc_sh, isem, gsem, ssem):
    c = lax.axis_index("c")
    s = lax.axis_index("s")
    wid = c * NS + s
    base = s * SEG
    pltpu.sync_copy(z_hbm.at[pl.ds(base, SEG)], acc_sh.at[pl.ds(base, SEG)])
    plsc.subcore_barrier()

    def idx_fetch(j, p):
        pltpu.async_copy(src_hbm.at[wid, j], sidx_v.at[p], isem)
        pltpu.async_copy(dst_hbm.at[wid, j, 0], didx_v.at[p, 0], isem)
        pltpu.async_copy(dst_hbm.at[wid, j, 1], didx_v.at[p, 1], isem)

    def idx_wait():
        pltpu.make_async_copy(src_hbm.at[0, 0], sidx_v.at[0], isem).wait()
        pltpu.make_async_copy(dst_hbm.at[0, 0, 0], didx_v.at[0, 0],
                              isem).wait()
        pltpu.make_async_copy(dst_hbm.at[0, 0, 0], didx_v.at[0, 0],
                              isem).wait()

    def gather(p):
        pltpu.async_copy(h_hbm.at[sidx_v.at[p]], rows_v.at[p], gsem)

    def gwait(p):
        pltpu.make_async_copy(h_hbm.at[sidx_v.at[0]], rows_v.at[p],
                              gsem).wait()

    def scatter(p):
        pltpu.async_copy(rows_v.at[p, pl.ds(0, CK // 2)],
                         acc_sh.at[didx_v.at[p, 0]], ssem, add=True)
        pltpu.async_copy(rows_v.at[p, pl.ds(CK // 2, CK // 2)],
                         acc_sh.at[didx_v.at[p, 1]], ssem, add=True)

    def swait(p):
        pltpu.make_async_copy(rows_v.at[p, pl.ds(0, CK // 2)],
                              acc_sh.at[didx_v.at[0, 0]], ssem).wait()
        pltpu.make_async_copy(rows_v.at[p, pl.ds(0, CK // 2)],
                              acc_sh.at[didx_v.at[0, 0]], ssem).wait()

    idx_fetch(0, 0)
    idx_wait()
    gather(0)
    idx_fetch(1, 1)
    gwait(0)
    idx_fetch(2, 2)
    idx_wait()
    gather(1)
    scatter(0)

    def half(g, p):
        gwait(p)
        swait((p + 2) % 3)
        idx_fetch(lax.rem(g + 2, CH), (p + 2) % 3)
        idx_wait()
        gather((p + 1) % 3)
        scatter(p)

    def body(i, carry):
        half(3 * i + 1, 1)
        half(3 * i + 2, 2)
        half(3 * i + 3, 0)
        return carry

    lax.fori_loop(0, (CH - 1) // 3, body, 0)
    swait((CH - 1) % 3)
    gwait(CH % 3)
    idx_wait()
    plsc.subcore_barrier()
    pltpu.sync_copy(acc_sh.at[pl.ds(base, SEG)],
                    out_hbm.at[c, pl.ds(base, SEG)])


_sc_scatter = pl.kernel(
    _sc_scatter_body,
    out_type=jax.ShapeDtypeStruct((NC, NACC, D), jnp.float32),
    mesh=plsc.VectorSubcoreMesh(core_axis_name="c", subcore_axis_name="s"),
    scratch_types=[
        pltpu.VMEM((3, CK), jnp.int32),
        pltpu.VMEM((3, 2, CK // 2), jnp.int32),
        pltpu.VMEM((3, CK, D), jnp.float32),
        pltpu.VMEM_SHARED((NACC, D), jnp.float32),
        pltpu.SemaphoreType.DMA,
        pltpu.SemaphoreType.DMA,
        pltpu.SemaphoreType.DMA,
    ],
)


def _tc_pre_body(llm_ref, ids_ref, wpt_ref, eemb_ref, bcat_ref, w1t_ref,
                 dinv_ref, x_ref, h_ref):
    xl = jnp.dot(llm_ref[...], wpt_ref[...], preferred_element_type=jnp.float32)
    oh = (ids_ref[...] == lax.broadcasted_iota(jnp.int32, (1, NSTRUCT), 1))
    xs = jnp.dot(oh.astype(jnp.float32), eemb_ref[...],
                 preferred_element_type=jnp.float32)
    x = xl + xs + bcat_ref[...]
    x_ref[...] = x
    h_ref[...] = jnp.dot(x, w1t_ref[...],
                         preferred_element_type=jnp.float32) * dinv_ref[...]


def _tc_mid_body(x_ref, h_ref, sp_ref, dinv_ref, b_ref, wt_ref, x2_ref, h2_ref):
    dinv = dinv_ref[...]
    out1 = (sp_ref[0] + sp_ref[1] + h_ref[...]) * dinv + b_ref[...]
    x2 = x_ref[...] + jnp.maximum(out1, 0.0)
    x2_ref[...] = x2
    h2_ref[...] = jnp.dot(x2, wt_ref[...],
                          preferred_element_type=jnp.float32) * dinv


def _tc_fin_body(x_ref, h_ref, sp_ref, dinv_ref, b_ref, g_ref, bt_ref, y_ref):
    out2 = (sp_ref[0] + sp_ref[1] + h_ref[...]) * dinv_ref[...] + b_ref[...]
    t = x_ref[...] + jnp.maximum(out2, 0.0)
    mu = jnp.mean(t, axis=1, keepdims=True)
    d = t - mu
    var = jnp.mean(d * d, axis=1, keepdims=True)
    y_ref[...] = d * lax.rsqrt(var + 1e-5) * g_ref[...] + bt_ref[...]


def _row_spec(cols):
    return pl.BlockSpec((BLK, cols), lambda i: (i, 0))


def _full2(r, c):
    return pl.BlockSpec((r, c), lambda i: (0, 0))


_SP_SPEC = pl.BlockSpec((NC, BLK, D), lambda i: (0, i, 0))

_tc_pre = pl.pallas_call(
    _tc_pre_body,
    grid=(GRID,),
    in_specs=[_row_spec(DLLM), _row_spec(1), _full2(DLLM, D),
              _full2(NSTRUCT, D), _full2(1, D), _full2(D, D), _row_spec(1)],
    out_specs=[_row_spec(D), _row_spec(D)],
    out_shape=[jax.ShapeDtypeStruct((N, D), jnp.float32),
               jax.ShapeDtypeStruct((N, D), jnp.float32)],
)

_tc_mid = pl.pallas_call(
    _tc_mid_body,
    grid=(GRID,),
    in_specs=[_row_spec(D), _row_spec(D), _SP_SPEC, _row_spec(1),
              _full2(1, D), _full2(D, D)],
    out_specs=[_row_spec(D), _row_spec(D)],
    out_shape=[jax.ShapeDtypeStruct((N, D), jnp.float32),
               jax.ShapeDtypeStruct((N, D), jnp.float32)],
)

_tc_fin = pl.pallas_call(
    _tc_fin_body,
    grid=(GRID,),
    in_specs=[_row_spec(D), _row_spec(D), _SP_SPEC, _row_spec(1),
              _full2(1, D), _full2(1, D), _full2(1, D)],
    out_specs=_row_spec(D),
    out_shape=jax.ShapeDtypeStruct((N, D), jnp.float32),
)


def kernel(llm_feat, struct_type_ids, edge_index, Wp, bp, Eemb,
           W1, b1, W2, b2, gamma, beta):
    f32 = jnp.float32
    src = edge_index[0].astype(jnp.int32)
    dst = edge_index[1].astype(jnp.int32)
    pad = EPAD - E
    pi = jnp.arange(pad, dtype=jnp.int32)
    srcp = jnp.concatenate([src, pi % N]).reshape(NW, CH, CK)
    dstp = jnp.concatenate([dst, N + pi % (NACC - N)]).reshape(NW, CH, 2,
                                                               CK // 2)
    dst_flat = dstp.reshape(NW, CH * CK)

    z2 = jnp.zeros((NACC, D), f32)

    degp = _sc_deg(dst_flat)
    deg = degp.sum(axis=0)[:N] + 1.0
    dinv = lax.rsqrt(deg).reshape(N, 1)

    ids = struct_type_ids.astype(jnp.int32).reshape(N, 1)
    wpt = jnp.zeros((DLLM, D), f32).at[:, :DPROJ].set(Wp.T)
    eemb_pad = jnp.zeros((NSTRUCT, D), f32).at[:, DPROJ:].set(Eemb)
    bcat = jnp.zeros((1, D), f32).at[0, :DPROJ].set(bp)

    xcat, h1 = _tc_pre(llm_feat, ids, wpt, eemb_pad, bcat, W1.T, dinv)
    s1 = _sc_scatter(h1, srcp, dstp, z2)
    x2, h2 = _tc_mid(xcat, h1, s1, dinv, b1.reshape(1, D), W2.T)
    s2 = _sc_scatter(h2, srcp, dstp, z2)
    return _tc_fin(x2, h2, s2, dinv, b2.reshape(1, D),
                   gamma.reshape(1, D), beta.reshape(1, D))


# speedup vs baseline: 1.0434x; 1.0434x over previous
import functools

import jax
import jax.numpy as jnp
from jax import lax
from jax.experimental import pallas as pl
from jax.experimental.pallas import tpu as pltpu
from jax.experimental.pallas import tpu_sc as plsc

N = 10000
E = 320000
D = 128
DLLM = 640
DPROJ = 112
NSTRUCT = 5
NC = 2
NS = 16
NW = NC * NS
CK = 100
CH = 100
EPAD = NW * CH * CK
NACC = 10112
SEG = NACC // NS
BLK = 1000
GRID = N // BLK


def _sc_deg_body(dst_hbm, deg_out, dst_v, deg_v):
    c = lax.axis_index("c")
    s = lax.axis_index("s")
    wid = c * NS + s
    pltpu.sync_copy(dst_hbm.at[wid], dst_v)

    zero16 = jnp.zeros((16,), jnp.float32)

    def zbody(i, carry):
        deg_v[pl.ds(i * 16, 16)] = zero16
        return carry

    lax.fori_loop(0, NACC // 16, zbody, 0)

    ones16 = jnp.ones((16,), jnp.float32)

    def ebody(k, carry):
        idx = dst_v[pl.ds(k * 16, 16)]
        plsc.addupdate_scatter(deg_v, [idx], ones16)
        return carry

    lax.fori_loop(0, (CH * CK) // 16, ebody, 0)
    pltpu.sync_copy(deg_v, deg_out.at[wid])


_sc_deg = pl.kernel(
    _sc_deg_body,
    out_type=jax.ShapeDtypeStruct((NW, NACC), jnp.float32),
    mesh=plsc.VectorSubcoreMesh(core_axis_name="c", subcore_axis_name="s"),
    scratch_types=[
        pltpu.VMEM((CH * CK,), jnp.int32),
        pltpu.VMEM((NACC,), jnp.float32),
    ],
    compiler_params=pltpu.CompilerParams(needs_layout_passes=False),
)


def _sc_scatter_body(h_hbm, src_hbm, dst_hbm, z_hbm, out_hbm,
                     sidx_v, didx_v, rows_v, acc_sh, isem, gsem, ssem):
    c = lax.axis_index("c")
    s = lax.axis_index("s")
    wid = c * NS + s
    base = s * SEG
    pltpu.sync_copy(z_hbm.at[pl.ds(base, SEG)], acc_sh.at[pl.ds(base, SEG)])
    plsc.subcore_barrier()

    def idx_fetch(j, p):
        pltpu.async_copy(src_hbm.at[wid, j], sidx_v.at[p], isem)
        pltpu.async_copy(dst_hbm.at[wid, j], didx_v.at[p], isem)

    def idx_wait():
        pltpu.make_async_copy(src_hbm.at[0, 0], sidx_v.at[0], isem).wait()
        pltpu.make_async_copy(src_hbm.at[0, 0], didx_v.at[0], isem).wait()

    def gather(p):
        pltpu.async_copy(h_hbm.at[sidx_v.at[p]], rows_v.at[p], gsem)

    def gwait(p):
        pltpu.make_async_copy(h_hbm.at[sidx_v.at[0]], rows_v.at[p],
                              gsem).wait()

    def scatter(p):
        pltpu.async_copy(rows_v.at[p], acc_sh.at[didx_v.at[p]], ssem,
                         add=True)

    def swait(p):
        pltpu.make_async_copy(rows_v.at[p], acc_sh.at[didx_v.at[0]],
                              ssem).wait()

    idx_fetch(0, 0)
    idx_wait()
    gather(0)
    idx_fetch(1, 1)
    gwait(0)
    idx_fetch(2, 2)
    idx_wait()
    gather(1)
    scatter(0)

    def half(g, p):
        gwait(p)
        swait((p + 2) % 3)
        idx_fetch(lax.rem(g + 2, CH), (p + 2) % 3)
        idx_wait()
        gather((p + 1) % 3)
        scatter(p)

    def body(i, carry):
        half(3 * i + 1, 1)
        half(3 * i + 2, 2)
        half(3 * i + 3, 0)
        return carry

    lax.fori_loop(0, (CH - 1) // 3, body, 0)
    swait((CH - 1) % 3)
    gwait(CH % 3)
    idx_wait()
    plsc.subcore_barrier()
    pltpu.sync_copy(acc_sh.at[pl.ds(base, SEG)],
                    out_hbm.at[c, pl.ds(base, SEG)])


_sc_scatter = pl.kernel(
    _sc_scatter_body,
    out_type=jax.ShapeDtypeStruct((NC, NACC, D), jnp.float32),
    mesh=plsc.VectorSubcoreMesh(core_axis_name="c", subcore_axis_name="s"),
    scratch_types=[
        pltpu.VMEM((3, CK), jnp.int32),
        pltpu.VMEM((3, CK), jnp.int32),
        pltpu.VMEM((3, CK, D), jnp.float32),
        pltpu.VMEM_SHARED((NACC, D), jnp.float32),
        pltpu.SemaphoreType.DMA,
        pltpu.SemaphoreType.DMA,
        pltpu.SemaphoreType.DMA,
    ],
)


def _tc_pre_body(llm_ref, ids_ref, wpt_ref, eemb_ref, bcat_ref, w1t_ref,
                 dinv_ref, x_ref, h_ref):
    xl = jnp.dot(llm_ref[...], wpt_ref[...], preferred_element_type=jnp.float32)
    oh = (ids_ref[...] == lax.broadcasted_iota(jnp.int32, (1, NSTRUCT), 1))
    xs = jnp.dot(oh.astype(jnp.float32), eemb_ref[...],
                 preferred_element_type=jnp.float32)
    x = xl + xs + bcat_ref[...]
    x_ref[...] = x
    h_ref[...] = jnp.dot(x, w1t_ref[...],
                         preferred_element_type=jnp.float32) * dinv_ref[...]


def _tc_mid_body(x_ref, h_ref, sp_ref, dinv_ref, b_ref, wt_ref, x2_ref, h2_ref):
    dinv = dinv_ref[...]
    out1 = (sp_ref[0] + sp_ref[1] + h_ref[...]) * dinv + b_ref[...]
    x2 = x_ref[...] + jnp.maximum(out1, 0.0)
    x2_ref[...] = x2
    h2_ref[...] = jnp.dot(x2, wt_ref[...],
                          preferred_element_type=jnp.float32) * dinv


def _tc_fin_body(x_ref, h_ref, sp_ref, dinv_ref, b_ref, g_ref, bt_ref, y_ref):
    out2 = (sp_ref[0] + sp_ref[1] + h_ref[...]) * dinv_ref[...] + b_ref[...]
    t = x_ref[...] + jnp.maximum(out2, 0.0)
    mu = jnp.mean(t, axis=1, keepdims=True)
    d = t - mu
    var = jnp.mean(d * d, axis=1, keepdims=True)
    y_ref[...] = d * lax.rsqrt(var + 1e-5) * g_ref[...] + bt_ref[...]


def _row_spec(cols):
    return pl.BlockSpec((BLK, cols), lambda i: (i, 0))


def _full2(r, c):
    return pl.BlockSpec((r, c), lambda i: (0, 0))


_SP_SPEC = pl.BlockSpec((NC, BLK, D), lambda i: (0, i, 0))

_tc_pre = pl.pallas_call(
    _tc_pre_body,
    grid=(GRID,),
    in_specs=[_row_spec(DLLM), _row_spec(1), _full2(DLLM, D),
              _full2(NSTRUCT, D), _full2(1, D), _full2(D, D), _row_spec(1)],
    out_specs=[_row_spec(D), _row_spec(D)],
    out_shape=[jax.ShapeDtypeStruct((N, D), jnp.float32),
               jax.ShapeDtypeStruct((N, D), jnp.float32)],
)

_tc_mid = pl.pallas_call(
    _tc_mid_body,
    grid=(GRID,),
    in_specs=[_row_spec(D), _row_spec(D), _SP_SPEC, _row_spec(1),
              _full2(1, D), _full2(D, D)],
    out_specs=[_row_spec(D), _row_spec(D)],
    out_shape=[jax.ShapeDtypeStruct((N, D), jnp.float32),
               jax.ShapeDtypeStruct((N, D), jnp.float32)],
)

_tc_fin = pl.pallas_call(
    _tc_fin_body,
    grid=(GRID,),
    in_specs=[_row_spec(D), _row_spec(D), _SP_SPEC, _row_spec(1),
              _full2(1, D), _full2(1, D), _full2(1, D)],
    out_specs=_row_spec(D),
    out_shape=jax.ShapeDtypeStruct((N, D), jnp.float32),
)


def kernel(llm_feat, struct_type_ids, edge_index, Wp, bp, Eemb,
           W1, b1, W2, b2, gamma, beta):
    f32 = jnp.float32
    src = edge_index[0].astype(jnp.int32)
    dst = edge_index[1].astype(jnp.int32)
    srcp = src.reshape(NW, CH, CK)
    dstp = dst.reshape(NW, CH, CK)
    dst_flat = dst.reshape(NW, CH * CK)

    z2 = jnp.zeros((NACC, D), f32)

    degp = _sc_deg(dst_flat)
    deg = degp.sum(axis=0)[:N] + 1.0
    dinv = lax.rsqrt(deg).reshape(N, 1)

    ids = struct_type_ids.astype(jnp.int32).reshape(N, 1)
    wpt = jnp.zeros((DLLM, D), f32).at[:, :DPROJ].set(Wp.T)
    eemb_pad = jnp.zeros((NSTRUCT, D), f32).at[:, DPROJ:].set(Eemb)
    bcat = jnp.zeros((1, D), f32).at[0, :DPROJ].set(bp)

    xcat, h1 = _tc_pre(llm_feat, ids, wpt, eemb_pad, bcat, W1.T, dinv)
    s1 = _sc_scatter(h1, srcp, dstp, z2)
    x2, h2 = _tc_mid(xcat, h1, s1, dinv, b1.reshape(1, D), W2.T)
    s2 = _sc_scatter(h2, srcp, dstp, z2)
    return _tc_fin(x2, h2, s2, dinv, b2.reshape(1, D),
                   gamma.reshape(1, D), beta.reshape(1, D))


# speedup vs baseline: 1.1067x; 1.0607x over previous
import functools

import jax
import jax.numpy as jnp
from jax import lax
from jax.experimental import pallas as pl
from jax.experimental.pallas import tpu as pltpu
from jax.experimental.pallas import tpu_sc as plsc

N = 10000
E = 320000
D = 128
DLLM = 640
DPROJ = 112
NSTRUCT = 5
NC = 2
NS = 16
NW = NC * NS
CK = 125
CH = 80
EPAD = NW * CH * CK
NACC = 10112
SEG = NACC // NS
BLK = 1000
GRID = N // BLK


def _sc_deg_body(dst_hbm, deg_out, dst_v, deg_v):
    c = lax.axis_index("c")
    s = lax.axis_index("s")
    wid = c * NS + s
    pltpu.sync_copy(dst_hbm.at[wid], dst_v)

    zero16 = jnp.zeros((16,), jnp.float32)

    def zbody(i, carry):
        deg_v[pl.ds(i * 16, 16)] = zero16
        return carry

    lax.fori_loop(0, NACC // 16, zbody, 0)

    ones16 = jnp.ones((16,), jnp.float32)

    def ebody(k, carry):
        idx = dst_v[pl.ds(k * 16, 16)]
        plsc.addupdate_scatter(deg_v, [idx], ones16)
        return carry

    lax.fori_loop(0, (CH * CK) // 16, ebody, 0)
    pltpu.sync_copy(deg_v, deg_out.at[wid])


_sc_deg = pl.kernel(
    _sc_deg_body,
    out_type=jax.ShapeDtypeStruct((NW, NACC), jnp.float32),
    mesh=plsc.VectorSubcoreMesh(core_axis_name="c", subcore_axis_name="s"),
    scratch_types=[
        pltpu.VMEM((CH * CK,), jnp.int32),
        pltpu.VMEM((NACC,), jnp.float32),
    ],
    compiler_params=pltpu.CompilerParams(needs_layout_passes=False),
)


def _sc_scatter_body(h_hbm, src_hbm, dst_hbm, z_hbm, out_hbm,
                     sidx_v, didx_v, rows_v, acc_sh, isem, gsem, ssem):
    c = lax.axis_index("c")
    s = lax.axis_index("s")
    wid = c * NS + s
    base = s * SEG
    pltpu.sync_copy(z_hbm.at[pl.ds(base, SEG)], acc_sh.at[pl.ds(base, SEG)])
    plsc.subcore_barrier()

    def idx_fetch(j, p):
        pltpu.async_copy(src_hbm.at[wid, j], sidx_v.at[p], isem)
        pltpu.async_copy(dst_hbm.at[wid, j], didx_v.at[p], isem)

    def idx_wait():
        pltpu.make_async_copy(src_hbm.at[0, 0], sidx_v.at[0], isem).wait()
        pltpu.make_async_copy(src_hbm.at[0, 0], didx_v.at[0], isem).wait()

    def gather(p):
        pltpu.async_copy(h_hbm.at[sidx_v.at[p]], rows_v.at[p], gsem)

    def gwait(p):
        pltpu.make_async_copy(h_hbm.at[sidx_v.at[0]], rows_v.at[p],
                              gsem).wait()

    def scatter(p):
        pltpu.async_copy(rows_v.at[p], acc_sh.at[didx_v.at[p]], ssem,
                         add=True)

    def swait(p):
        pltpu.make_async_copy(rows_v.at[p], acc_sh.at[didx_v.at[0]],
                              ssem).wait()

    idx_fetch(0, 0)
    idx_wait()
    gather(0)
    idx_fetch(1, 1)
    gwait(0)
    idx_fetch(2, 2)
    idx_wait()
    gather(1)
    scatter(0)

    def half(g, p):
        gwait(p)
        swait((p + 2) % 3)
        idx_fetch(lax.rem(g + 2, CH), (p + 2) % 3)
        idx_wait()
        gather((p + 1) % 3)
        scatter(p)

    def body(i, carry):
        half(3 * i + 1, 1)
        half(3 * i + 2, 2)
        half(3 * i + 3, 0)
        return carry

    lax.fori_loop(0, (CH - 2) // 3, body, 0)
    gwait((CH - 1) % 3)
    swait((CH - 2) % 3)
    scatter((CH - 1) % 3)
    swait((CH - 1) % 3)
    idx_wait()
    plsc.subcore_barrier()
    pltpu.sync_copy(acc_sh.at[pl.ds(base, SEG)],
                    out_hbm.at[c, pl.ds(base, SEG)])


_sc_scatter = pl.kernel(
    _sc_scatter_body,
    out_type=jax.ShapeDtypeStruct((NC, NACC, D), jnp.float32),
    mesh=plsc.VectorSubcoreMesh(core_axis_name="c", subcore_axis_name="s"),
    scratch_types=[
        pltpu.VMEM((3, CK), jnp.int32),
        pltpu.VMEM((3, CK), jnp.int32),
        pltpu.VMEM((3, CK, D), jnp.float32),
        pltpu.VMEM_SHARED((NACC, D), jnp.float32),
        pltpu.SemaphoreType.DMA,
        pltpu.SemaphoreType.DMA,
        pltpu.SemaphoreType.DMA,
    ],
)


def _tc_pre_body(llm_ref, ids_ref, wpt_ref, eemb_ref, bcat_ref, w1t_ref,
                 dinv_ref, x_ref, h_ref):
    xl = jnp.dot(llm_ref[...], wpt_ref[...], preferred_element_type=jnp.float32)
    oh = (ids_ref[...] == lax.broadcasted_iota(jnp.int32, (1, NSTRUCT), 1))
    xs = jnp.dot(oh.astype(jnp.float32), eemb_ref[...],
                 preferred_element_type=jnp.float32)
    x = xl + xs + bcat_ref[...]
    x_ref[...] = x
    h_ref[...] = jnp.dot(x, w1t_ref[...],
                         preferred_element_type=jnp.float32) * dinv_ref[...]


def _tc_mid_body(x_ref, h_ref, sp_ref, dinv_ref, b_ref, wt_ref, x2_ref, h2_ref):
    dinv = dinv_ref[...]
    out1 = (sp_ref[0] + sp_ref[1] + h_ref[...]) * dinv + b_ref[...]
    x2 = x_ref[...] + jnp.maximum(out1, 0.0)
    x2_ref[...] = x2
    h2_ref[...] = jnp.dot(x2, wt_ref[...],
                          preferred_element_type=jnp.float32) * dinv


def _tc_fin_body(x_ref, h_ref, sp_ref, dinv_ref, b_ref, g_ref, bt_ref, y_ref):
    out2 = (sp_ref[0] + sp_ref[1] + h_ref[...]) * dinv_ref[...] + b_ref[...]
    t = x_ref[...] + jnp.maximum(out2, 0.0)
    mu = jnp.mean(t, axis=1, keepdims=True)
    d = t - mu
    var = jnp.mean(d * d, axis=1, keepdims=True)
    y_ref[...] = d * lax.rsqrt(var + 1e-5) * g_ref[...] + bt_ref[...]


def _row_spec(cols):
    return pl.BlockSpec((BLK, cols), lambda i: (i, 0))


def _full2(r, c):
    return pl.BlockSpec((r, c), lambda i: (0, 0))


_SP_SPEC = pl.BlockSpec((NC, BLK, D), lambda i: (0, i, 0))

_tc_pre = pl.pallas_call(
    _tc_pre_body,
    grid=(GRID,),
    in_specs=[_row_spec(DLLM), _row_spec(1), _full2(DLLM, D),
              _full2(NSTRUCT, D), _full2(1, D), _full2(D, D), _row_spec(1)],
    out_specs=[_row_spec(D), _row_spec(D)],
    out_shape=[jax.ShapeDtypeStruct((N, D), jnp.float32),
               jax.ShapeDtypeStruct((N, D), jnp.float32)],
)

_tc_mid = pl.pallas_call(
    _tc_mid_body,
    grid=(GRID,),
    in_specs=[_row_spec(D), _row_spec(D), _SP_SPEC, _row_spec(1),
              _full2(1, D), _full2(D, D)],
    out_specs=[_row_spec(D), _row_spec(D)],
    out_shape=[jax.ShapeDtypeStruct((N, D), jnp.float32),
               jax.ShapeDtypeStruct((N, D), jnp.float32)],
)

_tc_fin = pl.pallas_call(
    _tc_fin_body,
    grid=(GRID,),
    in_specs=[_row_spec(D), _row_spec(D), _SP_SPEC, _row_spec(1),
              _full2(1, D), _full2(1, D), _full2(1, D)],
    out_specs=_row_spec(D),
    out_shape=jax.ShapeDtypeStruct((N, D), jnp.float32),
)


def kernel(llm_feat, struct_type_ids, edge_index, Wp, bp, Eemb,
           W1, b1, W2, b2, gamma, beta):
    f32 = jnp.float32
    src = edge_index[0].astype(jnp.int32)
    dst = edge_index[1].astype(jnp.int32)
    srcp = src.reshape(NW, CH, CK)
    dstp = dst.reshape(NW, CH, CK)
    dst_flat = dst.reshape(NW, CH * CK)

    z2 = jnp.zeros((NACC, D), f32)

    degp = _sc_deg(dst_flat)
    deg = degp.sum(axis=0)[:N] + 1.0
    dinv = lax.rsqrt(deg).reshape(N, 1)

    ids = struct_type_ids.astype(jnp.int32).reshape(N, 1)
    wpt = jnp.zeros((DLLM, D), f32).at[:, :DPROJ].set(Wp.T)
    eemb_pad = jnp.zeros((NSTRUCT, D), f32).at[:, DPROJ:].set(Eemb)
    bcat = jnp.zeros((1, D), f32).at[0, :DPROJ].set(bp)

    xcat, h1 = _tc_pre(llm_feat, ids, wpt, eemb_pad, bcat, W1.T, dinv)
    s1 = _sc_scatter(h1, srcp, dstp, z2)
    x2, h2 = _tc_mid(xcat, h1, s1, dinv, b1.reshape(1, D), W2.T)
    s2 = _sc_scatter(h2, srcp, dstp, z2)
    return _tc_fin(x2, h2, s2, dinv, b2.reshape(1, D),
                   gamma.reshape(1, D), beta.reshape(1, D))


# speedup vs baseline: 1.1258x; 1.0173x over previous
import functools

import jax
import jax.numpy as jnp
from jax import lax
from jax.experimental import pallas as pl
from jax.experimental.pallas import tpu as pltpu
from jax.experimental.pallas import tpu_sc as plsc

N = 10000
E = 320000
D = 128
DLLM = 640
DPROJ = 112
NSTRUCT = 5
NC = 2
NS = 16
NW = NC * NS
CK = 128
CH = 79
EPAD = NW * CH * CK
NACC = 10112
SEG = NACC // NS
BLK = 1000
GRID = N // BLK


def _sc_deg_body(dst_hbm, deg_out, dst_v, deg_v):
    c = lax.axis_index("c")
    s = lax.axis_index("s")
    wid = c * NS + s
    pltpu.sync_copy(dst_hbm.at[wid], dst_v)

    zero16 = jnp.zeros((16,), jnp.float32)

    def zbody(i, carry):
        deg_v[pl.ds(i * 16, 16)] = zero16
        return carry

    lax.fori_loop(0, NACC // 16, zbody, 0)

    ones16 = jnp.ones((16,), jnp.float32)

    def ebody(k, carry):
        idx = dst_v[pl.ds(k * 16, 16)]
        plsc.addupdate_scatter(deg_v, [idx], ones16)
        return carry

    lax.fori_loop(0, (CH * CK) // 16, ebody, 0)
    pltpu.sync_copy(deg_v, deg_out.at[wid])


_sc_deg = pl.kernel(
    _sc_deg_body,
    out_type=jax.ShapeDtypeStruct((NW, NACC), jnp.float32),
    mesh=plsc.VectorSubcoreMesh(core_axis_name="c", subcore_axis_name="s"),
    scratch_types=[
        pltpu.VMEM((CH * CK,), jnp.int32),
        pltpu.VMEM((NACC,), jnp.float32),
    ],
    compiler_params=pltpu.CompilerParams(needs_layout_passes=False),
)


def _sc_scatter_body(h_hbm, src_hbm, dst_hbm, z_hbm, out_hbm,
                     sidx_v, didx_v, rows_v, acc_sh, isem, gsem, ssem):
    c = lax.axis_index("c")
    s = lax.axis_index("s")
    wid = c * NS + s
    base = s * SEG
    pltpu.sync_copy(z_hbm.at[pl.ds(base, SEG)], acc_sh.at[pl.ds(base, SEG)])
    plsc.subcore_barrier()

    def idx_fetch(j, p):
        pltpu.async_copy(src_hbm.at[wid, j], sidx_v.at[p], isem)
        pltpu.async_copy(dst_hbm.at[wid, j], didx_v.at[p], isem)

    def idx_wait():
        pltpu.make_async_copy(src_hbm.at[0, 0], sidx_v.at[0], isem).wait()
        pltpu.make_async_copy(src_hbm.at[0, 0], didx_v.at[0], isem).wait()

    def gather(p):
        pltpu.async_copy(h_hbm.at[sidx_v.at[p]], rows_v.at[p], gsem)

    def gwait(p):
        pltpu.make_async_copy(h_hbm.at[sidx_v.at[0]], rows_v.at[p],
                              gsem).wait()

    def scatter(p):
        pltpu.async_copy(rows_v.at[p], acc_sh.at[didx_v.at[p]], ssem,
                         add=True)

    def swait(p):
        pltpu.make_async_copy(rows_v.at[p], acc_sh.at[didx_v.at[0]],
                              ssem).wait()

    idx_fetch(0, 0)
    idx_wait()
    gather(0)
    idx_fetch(1, 1)
    gwait(0)
    idx_fetch(2, 2)
    idx_wait()
    gather(1)
    scatter(0)

    def half(g, p):
        gwait(p)
        swait((p + 2) % 3)
        idx_fetch(lax.rem(g + 2, CH), (p + 2) % 3)
        idx_wait()
        gather((p + 1) % 3)
        scatter(p)

    def body(i, carry):
        half(3 * i + 1, 1)
        half(3 * i + 2, 2)
        half(3 * i + 3, 0)
        return carry

    lax.fori_loop(0, (CH - 1) // 3, body, 0)
    swait((CH - 1) % 3)
    gwait(CH % 3)
    idx_wait()
    plsc.subcore_barrier()
    pltpu.sync_copy(acc_sh.at[pl.ds(base, SEG)],
                    out_hbm.at[c, pl.ds(base, SEG)])


_sc_scatter = pl.kernel(
    _sc_scatter_body,
    out_type=jax.ShapeDtypeStruct((NC, NACC, D), jnp.float32),
    mesh=plsc.VectorSubcoreMesh(core_axis_name="c", subcore_axis_name="s"),
    scratch_types=[
        pltpu.VMEM((3, CK), jnp.int32),
        pltpu.VMEM((3, CK), jnp.int32),
        pltpu.VMEM((3, CK, D), jnp.float32),
        pltpu.VMEM_SHARED((NACC, D), jnp.float32),
        pltpu.SemaphoreType.DMA,
        pltpu.SemaphoreType.DMA,
        pltpu.SemaphoreType.DMA,
    ],
)


def _tc_pre_body(llm_ref, ids_ref, wpt_ref, eemb_ref, bcat_ref, w1t_ref,
                 dinv_ref, x_ref, h_ref):
    xl = jnp.dot(llm_ref[...], wpt_ref[...], preferred_element_type=jnp.float32)
    oh = (ids_ref[...] == lax.broadcasted_iota(jnp.int32, (1, NSTRUCT), 1))
    xs = jnp.dot(oh.astype(jnp.float32), eemb_ref[...],
                 preferred_element_type=jnp.float32)
    x = xl + xs + bcat_ref[...]
    x_ref[...] = x
    h_ref[...] = jnp.dot(x, w1t_ref[...],
                         preferred_element_type=jnp.float32) * dinv_ref[...]


def _tc_mid_body(x_ref, h_ref, sp_ref, dinv_ref, b_ref, wt_ref, x2_ref, h2_ref):
    dinv = dinv_ref[...]
    out1 = (sp_ref[0] + sp_ref[1] + h_ref[...]) * dinv + b_ref[...]
    x2 = x_ref[...] + jnp.maximum(out1, 0.0)
    x2_ref[...] = x2
    h2_ref[...] = jnp.dot(x2, wt_ref[...],
                          preferred_element_type=jnp.float32) * dinv


def _tc_fin_body(x_ref, h_ref, sp_ref, dinv_ref, b_ref, g_ref, bt_ref, y_ref):
    out2 = (sp_ref[0] + sp_ref[1] + h_ref[...]) * dinv_ref[...] + b_ref[...]
    t = x_ref[...] + jnp.maximum(out2, 0.0)
    mu = jnp.mean(t, axis=1, keepdims=True)
    d = t - mu
    var = jnp.mean(d * d, axis=1, keepdims=True)
    y_ref[...] = d * lax.rsqrt(var + 1e-5) * g_ref[...] + bt_ref[...]


def _row_spec(cols):
    return pl.BlockSpec((BLK, cols), lambda i: (i, 0))


def _full2(r, c):
    return pl.BlockSpec((r, c), lambda i: (0, 0))


_SP_SPEC = pl.BlockSpec((NC, BLK, D), lambda i: (0, i, 0))

_tc_pre = pl.pallas_call(
    _tc_pre_body,
    grid=(GRID,),
    in_specs=[_row_spec(DLLM), _row_spec(1), _full2(DLLM, D),
              _full2(NSTRUCT, D), _full2(1, D), _full2(D, D), _row_spec(1)],
    out_specs=[_row_spec(D), _row_spec(D)],
    out_shape=[jax.ShapeDtypeStruct((N, D), jnp.float32),
               jax.ShapeDtypeStruct((N, D), jnp.float32)],
)

_tc_mid = pl.pallas_call(
    _tc_mid_body,
    grid=(GRID,),
    in_specs=[_row_spec(D), _row_spec(D), _SP_SPEC, _row_spec(1),
              _full2(1, D), _full2(D, D)],
    out_specs=[_row_spec(D), _row_spec(D)],
    out_shape=[jax.ShapeDtypeStruct((N, D), jnp.float32),
               jax.ShapeDtypeStruct((N, D), jnp.float32)],
)

_tc_fin = pl.pallas_call(
    _tc_fin_body,
    grid=(GRID,),
    in_specs=[_row_spec(D), _row_spec(D), _SP_SPEC, _row_spec(1),
              _full2(1, D), _full2(1, D), _full2(1, D)],
    out_specs=_row_spec(D),
    out_shape=jax.ShapeDtypeStruct((N, D), jnp.float32),
)


def kernel(llm_feat, struct_type_ids, edge_index, Wp, bp, Eemb,
           W1, b1, W2, b2, gamma, beta):
    f32 = jnp.float32
    src = edge_index[0].astype(jnp.int32)
    dst = edge_index[1].astype(jnp.int32)
    pad = EPAD - E
    pi = jnp.arange(pad, dtype=jnp.int32)
    srcp = jnp.concatenate([src, pi % N]).reshape(NW, CH, CK)
    dstp = jnp.concatenate([dst, N + pi % (NACC - N)]).reshape(NW, CH, CK)
    dst_flat = dstp.reshape(NW, CH * CK)

    z2 = jnp.zeros((NACC, D), f32)

    degp = _sc_deg(dst_flat)
    deg = degp.sum(axis=0)[:N] + 1.0
    dinv = lax.rsqrt(deg).reshape(N, 1)

    ids = struct_type_ids.astype(jnp.int32).reshape(N, 1)
    wpt = jnp.zeros((DLLM, D), f32).at[:, :DPROJ].set(Wp.T)
    eemb_pad = jnp.zeros((NSTRUCT, D), f32).at[:, DPROJ:].set(Eemb)
    bcat = jnp.zeros((1, D), f32).at[0, :DPROJ].set(bp)

    xcat, h1 = _tc_pre(llm_feat, ids, wpt, eemb_pad, bcat, W1.T, dinv)
    s1 = _sc_scatter(h1, srcp, dstp, z2)
    x2, h2 = _tc_mid(xcat, h1, s1, dinv, b1.reshape(1, D), W2.T)
    s2 = _sc_scatter(h2, srcp, dstp, z2)
    return _tc_fin(x2, h2, s2, dinv, b2.reshape(1, D),
                   gamma.reshape(1, D), beta.reshape(1, D))
